# Initial kernel scaffold; baseline (speedup 1.0000x reference)
import jax
import jax.numpy as jnp
from jax.experimental import pallas as pl


def kernel(input_ids, embed_table, W, b):
    raise NotImplementedError("write your pallas kernel here")


# speedup vs baseline: 2.4028x; 2.4028x over previous
import functools

import jax
import jax.numpy as jnp
from jax import lax
from jax.experimental import pallas as pl
from jax.experimental.pallas import tpu as pltpu
from jax.experimental.pallas import tpu_sc as plsc

B = 4096
S = 200
H = 32
NL = 2
NC = 2
NS = 16
NW = NC * NS
BPW = B // NW
GL = 100
G = BPW * S // GL
CH_EX = 4
CH_ROWS = CH_EX * S
CH_G = CH_ROWS // GL
NCH = BPW // CH_EX


def _sc_pool(ids_r, table):
    mesh = plsc.VectorSubcoreMesh(
        core_axis_name="c", subcore_axis_name="s",
        num_cores=NC, num_subcores=NS)

    @functools.partial(
        pl.kernel, mesh=mesh,
        compiler_params=pltpu.CompilerParams(use_tc_tiling_on_sc=False),
        out_type=jax.ShapeDtypeStruct((B, H), jnp.float32),
        scratch_types=[
            pltpu.VMEM((G, GL), jnp.int32),
            pltpu.VMEM((CH_ROWS, H), jnp.float32),
            pltpu.VMEM((CH_ROWS, H), jnp.float32),
            pltpu.VMEM((BPW, H), jnp.float32),
            pltpu.SemaphoreType.DMA,
            pltpu.SemaphoreType.DMA,
        ],
    )
    def pool(ids_hbm, table_hbm, out_hbm, idx_v, rows0, rows1, acc_v,
             sem0, sem1):
        w = lax.axis_index("s") * NC + lax.axis_index("c")
        rows = (rows0, rows1)
        sems = (sem0, sem1)

        pltpu.sync_copy(ids_hbm.at[w], idx_v)

        def fire(ch, bi):
            for g in range(CH_G):
                pltpu.async_copy(
                    table_hbm.at[idx_v.at[ch * CH_G + g]],
                    rows[bi].at[pl.ds(g * GL, GL)],
                    sems[bi])

        def drain(bi):
            pltpu.make_async_copy(
                table_hbm.at[pl.ds(0, CH_ROWS)], rows[bi], sems[bi]).wait()

        def consume(ch, bi):
            for e in range(CH_EX):
                base = e * S

                def red(j, carry, _base=base, _bi=bi):
                    a0, a1 = carry
                    r = _base + j * 4
                    for u in range(4):
                        a0 = a0 + rows[_bi][r + u, 0:16]
                        a1 = a1 + rows[_bi][r + u, 16:32]
                    return a0, a1

                z = jnp.zeros((16,), jnp.float32)
                a0, a1 = lax.fori_loop(0, S // 4, red, (z, z))
                ex = ch * CH_EX + e
                acc_v[ex, 0:16] = a0
                acc_v[ex, 16:32] = a1

        fire(0, 0)
        fire(1, 1)

        def step(i, carry):
            ci = i * 2
            for bi in range(2):
                ch = ci + bi
                drain(bi)
                consume(ch, bi)

                @pl.when(ch + 2 < NCH)
                def _(ch=ch, bi=bi):
                    fire(ch + 2, bi)
            return carry

        lax.fori_loop(0, NCH // 2, step, 0)
        pltpu.sync_copy(acc_v, out_hbm.at[pl.ds(w * BPW, BPW)])

    return pool(ids_r, table)


def _tc_head(x, w2, b2):
    def body(x_ref, w_ref, b_ref, o_ref):
        o_ref[...] = jnp.dot(
            x_ref[...], w_ref[...],
            preferred_element_type=jnp.float32) + b_ref[...]

    return pl.pallas_call(
        body,
        out_shape=jax.ShapeDtypeStruct((B, NL), jnp.float32),
    )(x, w2, b2)


def kernel(input_ids, embed_table, W, b):
    ids_r = input_ids.reshape(NW, BPW, S).reshape(NW, G, GL)
    sums = _sc_pool(ids_r, embed_table)
    w2 = W * jnp.float32(1.0 / S)
    return _tc_head(sums, w2, b.reshape(1, NL))


# speedup vs baseline: 6.9378x; 2.8874x over previous
import functools

import jax
import jax.numpy as jnp
from jax import lax
from jax.experimental import pallas as pl
from jax.experimental.pallas import tpu as pltpu
from jax.experimental.pallas import tpu_sc as plsc

B = 4096
S = 200
H = 32
NL = 2
NC = 2
NS = 16
NW = NC * NS
BPW = B // NW
GL = 100
G = BPW * S // GL
CH_EX = 4
CH_ROWS = CH_EX * S
CH_G = CH_ROWS // GL
NCH = BPW // CH_EX


def _sc_pool(ids_r, table):
    mesh = plsc.VectorSubcoreMesh(
        core_axis_name="c", subcore_axis_name="s",
        num_cores=NC, num_subcores=NS)

    @functools.partial(
        pl.kernel, mesh=mesh,
        compiler_params=pltpu.CompilerParams(use_tc_tiling_on_sc=False),
        out_type=jax.ShapeDtypeStruct((B, H), jnp.float32),
        scratch_types=[
            pltpu.VMEM((G, GL), jnp.int32),
            pltpu.VMEM((CH_ROWS, H), jnp.float32),
            pltpu.VMEM((CH_ROWS, H), jnp.float32),
            pltpu.VMEM((BPW, H), jnp.float32),
            pltpu.SemaphoreType.DMA,
            pltpu.SemaphoreType.DMA,
        ],
    )
    def pool(ids_hbm, table_hbm, out_hbm, idx_v, rows0, rows1, acc_v,
             sem0, sem1):
        w = lax.axis_index("s") * NC + lax.axis_index("c")
        rows = (rows0, rows1)
        sems = (sem0, sem1)

        pltpu.sync_copy(ids_hbm.at[w], idx_v)

        def fire(ch, bi):
            for g in range(CH_G):
                pltpu.async_copy(
                    table_hbm.at[idx_v.at[ch * CH_G + g]],
                    rows[bi].at[pl.ds(g * GL, GL)],
                    sems[bi])

        def drain(bi):
            pltpu.make_async_copy(
                table_hbm.at[pl.ds(0, CH_ROWS)], rows[bi], sems[bi]).wait()

        def consume(ch, bi):
            for e in range(CH_EX):
                base = e * S

                def red(j, carry, _base=base, _bi=bi):
                    a0, a1 = carry
                    r = _base + j * 4
                    for u in range(4):
                        a0 = a0 + rows[_bi][r + u, 0:16]
                        a1 = a1 + rows[_bi][r + u, 16:32]
                    return a0, a1

                z = jnp.zeros((16,), jnp.float32)
                a0, a1 = lax.fori_loop(0, S // 4, red, (z, z))
                ex = ch * CH_EX + e
                acc_v[ex, 0:16] = a0
                acc_v[ex, 16:32] = a1

        fire(0, 0)
        fire(1, 1)

        def step(i, carry):
            ci = i * 2
            for bi in range(2):
                ch = ci + bi
                drain(bi)
                consume(ch, bi)

                @pl.when(ch + 2 < NCH)
                def _(ch=ch, bi=bi):
                    fire(ch + 2, bi)
            return carry

        lax.fori_loop(0, NCH // 2, step, 0)
        pltpu.sync_copy(acc_v, out_hbm.at[pl.ds(w * BPW, BPW)])

    return pool(ids_r, table)


TC = 32768
TC4 = TC // 4
TNB = 31
VP = TNB * TC


def _tc_transpose(table_t):
    V = table_t.shape[1]

    def body(x_ref, eye_ref, o_ref):
        i = pl.program_id(0)
        x = x_ref[...]

        def masked(v):
            col = jax.lax.broadcasted_iota(jnp.int32, (H, TC), 1)
            return jnp.where(col + i * TC < V, v, 0.0)

        x = jax.lax.cond(i == TNB - 1, masked, lambda v: v, x)
        xs = jnp.concatenate(
            [x[:, m * TC4:(m + 1) * TC4] for m in range(4)], axis=0)
        o_ref[...] = jax.lax.dot_general(
            xs, eye_ref[...], (((0,), (0,)), ((), ())),
            preferred_element_type=jnp.float32)

    eye = jnp.eye(4 * H, dtype=jnp.float32)
    return pl.pallas_call(
        body,
        grid=(TNB,),
        in_specs=[pl.BlockSpec((H, TC), lambda i: (0, i)),
                  pl.BlockSpec((4 * H, 4 * H), lambda i: (0, 0))],
        out_specs=pl.BlockSpec((TC4, 4 * H), lambda i: (i, 0)),
        out_shape=jax.ShapeDtypeStruct((VP // 4, 4 * H), jnp.float32),
    )(table_t, eye)


def _tc_head(x, w2, b2):
    def body(x_ref, w_ref, b_ref, o_ref):
        o_ref[...] = jnp.dot(
            x_ref[...], w_ref[...],
            preferred_element_type=jnp.float32) + b_ref[...]

    return pl.pallas_call(
        body,
        out_shape=jax.ShapeDtypeStruct((B, NL), jnp.float32),
    )(x, w2, b2)


def kernel(input_ids, embed_table, W, b):
    r = input_ids
    s = ((r >> 15) << 15) | ((r & 0x1FFF) << 2) | ((r >> 13) & 3)
    ids_r = s.reshape(NW, BPW, S).reshape(NW, G, GL)
    packed = _tc_transpose(embed_table.T)
    table_rm = packed.reshape(VP, H)
    sums = _sc_pool(ids_r, table_rm)
    w2 = W * jnp.float32(1.0 / S)
    return _tc_head(sums, w2, b.reshape(1, NL))


# speedup vs baseline: 7.5910x; 1.0941x over previous
import functools

import jax
import jax.numpy as jnp
from jax import lax
from jax.experimental import pallas as pl
from jax.experimental.pallas import tpu as pltpu
from jax.experimental.pallas import tpu_sc as plsc

B = 4096
S = 200
H = 32
NL = 2
NC = 2
NS = 16
NW = NC * NS
BPW = B // NW
GL = 100
G = BPW * S // GL
CH_EX = 8
CH_ROWS = CH_EX * S
CH_G = CH_ROWS // GL
NCH = BPW // CH_EX

TC = 32768
TC8 = TC // 8
TNB = 31
VP = TNB * TC


def _sc_pool(ids_r, table):
    mesh = plsc.VectorSubcoreMesh(
        core_axis_name="c", subcore_axis_name="s",
        num_cores=NC, num_subcores=NS)

    @functools.partial(
        pl.kernel, mesh=mesh,
        compiler_params=pltpu.CompilerParams(
            use_tc_tiling_on_sc=False, needs_layout_passes=False),
        out_type=jax.ShapeDtypeStruct((B, H), jnp.float32),
        scratch_types=[
            pltpu.VMEM((G, GL), jnp.int32),
            pltpu.VMEM((CH_ROWS, 16), jnp.int32),
            pltpu.VMEM((CH_ROWS, 16), jnp.int32),
            pltpu.VMEM((BPW, H), jnp.float32),
            pltpu.SemaphoreType.DMA,
            pltpu.SemaphoreType.DMA,
        ],
    )
    def pool(ids_hbm, table_hbm, out_hbm, idx_v, rows0, rows1, acc_v,
             sem0, sem1):
        w = lax.axis_index("s") * NC + lax.axis_index("c")
        rows = (rows0, rows1)
        sems = (sem0, sem1)

        pltpu.sync_copy(ids_hbm.at[w], idx_v)

        def fire(ch, bi):
            for g in range(CH_G):
                pltpu.async_copy(
                    table_hbm.at[idx_v.at[ch * CH_G + g]],
                    rows[bi].at[pl.ds(g * GL, GL)],
                    sems[bi])

        def drain(bi):
            pltpu.make_async_copy(
                table_hbm.at[pl.ds(0, CH_ROWS)], rows[bi], sems[bi]).wait()

        hi_mask = jnp.full((16,), -65536, jnp.int32)

        def consume(ch, bi):
            for e in range(CH_EX):
                base = e * S

                def red(j, carry, _base=base, _bi=bi):
                    a_lo, a_hi = carry
                    r = _base + j * 4
                    for u in range(4):
                        x = rows[_bi][r + u, 0:16]
                        a_lo = a_lo + plsc.bitcast(
                            x << 16, jnp.float32)
                        a_hi = a_hi + plsc.bitcast(
                            x & hi_mask, jnp.float32)
                    return a_lo, a_hi

                z = jnp.zeros((16,), jnp.float32)
                a_lo, a_hi = lax.fori_loop(0, S // 4, red, (z, z))
                ex = ch * CH_EX + e
                acc_v[ex, 0:16] = a_lo
                acc_v[ex, 16:32] = a_hi

        fire(0, 0)
        fire(1, 1)

        def step(i, carry):
            ci = i * 2
            for bi in range(2):
                ch = ci + bi
                drain(bi)
                consume(ch, bi)

                @pl.when(ch + 2 < NCH)
                def _(ch=ch, bi=bi):
                    fire(ch + 2, bi)
            return carry

        lax.fori_loop(0, NCH // 2, step, 0)
        pltpu.sync_copy(acc_v, out_hbm.at[pl.ds(w * BPW, BPW)])

    return pool(ids_r, table)


def _tc_transpose(table_t):
    V = table_t.shape[1]

    def body(x_ref, eye_ref, o_ref):
        i = pl.program_id(0)
        x = x_ref[...]

        def masked(v):
            col = jax.lax.broadcasted_iota(jnp.int32, (H, TC), 1)
            return jnp.where(col + i * TC < V, v, 0.0)

        x = jax.lax.cond(i == TNB - 1, masked, lambda v: v, x)
        xs_lo = jnp.concatenate(
            [x[0:16, m * TC8:(m + 1) * TC8] for m in range(8)], axis=0)
        xs_hi = jnp.concatenate(
            [x[16:32, m * TC8:(m + 1) * TC8] for m in range(8)], axis=0)
        eye = eye_ref[...]
        tl = jax.lax.dot_general(
            xs_lo, eye, (((0,), (0,)), ((), ())),
            preferred_element_type=jnp.float32)
        th = jax.lax.dot_general(
            xs_hi, eye, (((0,), (0,)), ((), ())),
            preferred_element_type=jnp.float32)
        lo = jax.lax.bitcast_convert_type(tl, jnp.int32)
        hi = jax.lax.bitcast_convert_type(th, jnp.int32)
        lo = jax.lax.shift_right_logical(lo + 0x8000, 16)
        hi = (hi + 0x8000) & jnp.int32(-65536)
        o_ref[...] = lo | hi

    eye = jnp.eye(4 * H, dtype=jnp.float32)
    return pl.pallas_call(
        body,
        grid=(TNB,),
        in_specs=[pl.BlockSpec((H, TC), lambda i: (0, i)),
                  pl.BlockSpec((4 * H, 4 * H), lambda i: (0, 0))],
        out_specs=pl.BlockSpec((TC8, 4 * H), lambda i: (i, 0)),
        out_shape=jax.ShapeDtypeStruct((VP // 8, 4 * H), jnp.int32),
    )(table_t, eye)


def _tc_head(x, w2, b2):
    def body(x_ref, w_ref, b_ref, o_ref):
        o_ref[...] = jnp.dot(
            x_ref[...], w_ref[...],
            preferred_element_type=jnp.float32) + b_ref[...]

    return pl.pallas_call(
        body,
        out_shape=jax.ShapeDtypeStruct((B, NL), jnp.float32),
    )(x, w2, b2)


def kernel(input_ids, embed_table, W, b):
    r = input_ids
    s = ((r >> 15) << 15) | ((r & 0xFFF) << 3) | ((r >> 12) & 7)
    ids_r = s.reshape(NW, BPW, S).reshape(NW, G, GL)
    packed = _tc_transpose(embed_table.T)
    table_rm = packed.reshape(VP, 16)
    sums = _sc_pool(ids_r, table_rm)
    w2 = W * jnp.float32(1.0 / S)
    return _tc_head(sums, w2, b.reshape(1, NL))


# speedup vs baseline: 8.6573x; 1.1405x over previous
import functools

import jax
import jax.numpy as jnp
from jax import lax
from jax.experimental import pallas as pl
from jax.experimental.pallas import tpu as pltpu
from jax.experimental.pallas import tpu_sc as plsc

B = 4096
S = 200
H = 32
NL = 2
NC = 2
NS = 16
NW = NC * NS
BPW = B // NW
GL = 100
G = BPW * S // GL
CH_EX = 8
CH_ROWS = CH_EX * S
CH_G = CH_ROWS // GL
NCH = BPW // CH_EX

TC = 32768
TC8 = TC // 8
TNB = 31
VP = TNB * TC


def _sc_pool(ids_r, table):
    mesh = plsc.VectorSubcoreMesh(
        core_axis_name="c", subcore_axis_name="s",
        num_cores=NC, num_subcores=NS)

    @functools.partial(
        pl.kernel, mesh=mesh,
        compiler_params=pltpu.CompilerParams(
            use_tc_tiling_on_sc=False, needs_layout_passes=False),
        out_type=jax.ShapeDtypeStruct((B, H), jnp.float32),
        scratch_types=[
            pltpu.VMEM((G, GL), jnp.int32),
            pltpu.VMEM((CH_ROWS, 16), jnp.int32),
            pltpu.VMEM((CH_ROWS, 16), jnp.int32),
            pltpu.VMEM((BPW, H), jnp.float32),
            pltpu.SemaphoreType.DMA,
            pltpu.SemaphoreType.DMA,
        ],
    )
    def pool(ids_hbm, table_hbm, out_hbm, idx_v, rows0, rows1, acc_v,
             sem0, sem1):
        w = lax.axis_index("s") * NC + lax.axis_index("c")
        rows = (rows0, rows1)
        sems = (sem0, sem1)

        pltpu.sync_copy(ids_hbm.at[w], idx_v)

        def fire(ch, bi):
            for g in range(CH_G):
                pltpu.async_copy(
                    table_hbm.at[idx_v.at[ch * CH_G + g]],
                    rows[bi].at[pl.ds(g * GL, GL)],
                    sems[bi])

        def drain(bi):
            pltpu.make_async_copy(
                table_hbm.at[pl.ds(0, CH_ROWS)], rows[bi], sems[bi]).wait()

        hi_mask = jnp.full((16,), -65536, jnp.int32)

        def consume(ch, bi):
            for e in range(CH_EX):
                base = e * S

                def red(j, carry, _base=base, _bi=bi):
                    a_lo, a_hi = carry
                    r = _base + j * 4
                    for u in range(4):
                        x = rows[_bi][r + u, 0:16]
                        a_lo = a_lo + plsc.bitcast(
                            x << 16, jnp.float32)
                        a_hi = a_hi + plsc.bitcast(
                            x & hi_mask, jnp.float32)
                    return a_lo, a_hi

                z = jnp.zeros((16,), jnp.float32)
                a_lo, a_hi = lax.fori_loop(0, S // 4, red, (z, z))
                ex = ch * CH_EX + e
                acc_v[ex, 0:16] = a_lo
                acc_v[ex, 16:32] = a_hi

        fire(0, 0)
        fire(1, 1)

        def step(i, carry):
            ci = i * 2
            for bi in range(2):
                ch = ci + bi
                drain(bi)
                consume(ch, bi)

                @pl.when(ch + 2 < NCH)
                def _(ch=ch, bi=bi):
                    fire(ch + 2, bi)
            return carry

        lax.fori_loop(0, NCH // 2, step, 0)
        pltpu.sync_copy(acc_v, out_hbm.at[pl.ds(w * BPW, BPW)])

    return pool(ids_r, table)


def _tc_transpose(table_t):
    V = table_t.shape[1]

    def body(x_ref, mask_ref, eye_ref, o_ref):
        xb = x_ref[...].astype(jnp.bfloat16) * mask_ref[0]
        xs_lo = jnp.concatenate(
            [xb[0:16, m * TC8:(m + 1) * TC8] for m in range(8)], axis=0)
        xs_hi = jnp.concatenate(
            [xb[16:32, m * TC8:(m + 1) * TC8] for m in range(8)], axis=0)
        eye = eye_ref[...]
        tl = jax.lax.dot_general(
            xs_lo, eye, (((0,), (0,)), ((), ())),
            preferred_element_type=jnp.float32)
        th = jax.lax.dot_general(
            xs_hi, eye, (((0,), (0,)), ((), ())),
            preferred_element_type=jnp.float32)
        lo = jax.lax.bitcast_convert_type(tl, jnp.int32)
        hi = jax.lax.bitcast_convert_type(th, jnp.int32)
        o_ref[...] = jax.lax.shift_right_logical(lo, 16) | hi

    eye = jnp.eye(4 * H, dtype=jnp.bfloat16)
    mask = (jnp.arange(TNB * TC, dtype=jnp.int32) < V).astype(
        jnp.bfloat16).reshape(TNB, 1, TC)
    return pl.pallas_call(
        body,
        grid=(TNB,),
        in_specs=[pl.BlockSpec((H, TC), lambda i: (0, i)),
                  pl.BlockSpec((1, 1, TC), lambda i: (i, 0, 0)),
                  pl.BlockSpec((4 * H, 4 * H), lambda i: (0, 0))],
        out_specs=pl.BlockSpec((TC8, 4 * H), lambda i: (i, 0)),
        out_shape=jax.ShapeDtypeStruct((VP // 8, 4 * H), jnp.int32),
    )(table_t, mask, eye)


def _tc_head(x, w2, b2):
    def body(x_ref, w_ref, b_ref, o_ref):
        o_ref[...] = jnp.dot(
            x_ref[...], w_ref[...],
            preferred_element_type=jnp.float32) + b_ref[...]

    return pl.pallas_call(
        body,
        out_shape=jax.ShapeDtypeStruct((B, NL), jnp.float32),
    )(x, w2, b2)


def kernel(input_ids, embed_table, W, b):
    r = input_ids
    s = ((r >> 15) << 15) | ((r & 0xFFF) << 3) | ((r >> 12) & 7)
    ids_r = s.reshape(NW, BPW, S).reshape(NW, G, GL)
    packed = _tc_transpose(embed_table.T)
    table_rm = packed.reshape(VP, 16)
    sums = _sc_pool(ids_r, table_rm)
    w2 = W * jnp.float32(1.0 / S)
    return _tc_head(sums, w2, b.reshape(1, NL))
